# baseline (speedup 1.0000x reference)
import functools

import jax
import jax.numpy as jnp
from jax import lax
from jax.experimental import pallas as pl
from jax.experimental.pallas import tpu as pltpu
from jax.experimental.pallas import tpu_sc as plsc

_NC = 2
_NS = 16
_NW = _NC * _NS


def _sc_copy_body(nodes_hbm, edges_hbm, out_nodes_hbm, out_edges_hbm):
    wid = lax.axis_index("s") * _NC + lax.axis_index("c")
    e_rows = edges_hbm.shape[0] // _NW
    e_base = wid * e_rows
    pltpu.sync_copy(edges_hbm.at[pl.ds(e_base, e_rows), :],
                    out_edges_hbm.at[pl.ds(e_base, e_rows), :])
    n_rows = 400
    n_workers = nodes_hbm.shape[0] // n_rows

    @pl.when(wid < n_workers)
    def _():
        n_base = wid * n_rows
        pltpu.sync_copy(nodes_hbm.at[pl.ds(n_base, n_rows), :],
                        out_nodes_hbm.at[pl.ds(n_base, n_rows), :])


def kernel(node_latents_from, node_latents_to, edge_latents, edge_index,
           receivers_count):
    del node_latents_from, edge_index, receivers_count
    mesh = plsc.VectorSubcoreMesh(
        core_axis_name="c", subcore_axis_name="s",
        num_cores=_NC, num_subcores=_NS)
    run = functools.partial(
        pl.kernel,
        out_type=(
            jax.ShapeDtypeStruct(node_latents_to.shape, node_latents_to.dtype),
            jax.ShapeDtypeStruct(edge_latents.shape, edge_latents.dtype),
        ),
        mesh=mesh,
    )(_sc_copy_body)
    return run(node_latents_to, edge_latents)


# speedup vs baseline: 16.8449x; 16.8449x over previous
import functools

import jax
import jax.numpy as jnp
from jax import lax
from jax.experimental import pallas as pl
from jax.experimental.pallas import tpu as pltpu
from jax.experimental.pallas import tpu_sc as plsc

_NC = 2
_NS = 16
_NW = _NC * _NS

_E_CHUNK = 200
_N_CHUNK = 80


def _sc_copy_body(nodes_hbm, edges_hbm, out_nodes_hbm, out_edges_hbm,
                  ebuf0, ebuf1, nbuf, sems_in, sems_out):
    wid = lax.axis_index("s") * _NC + lax.axis_index("c")
    e_rows = edges_hbm.shape[0] // _NW
    e_base = pl.multiple_of(wid * e_rows, 8)
    n_chunks = e_rows // _E_CHUNK
    ebufs = (ebuf0, ebuf1)

    def in_copy(i, slot):
        return pltpu.make_async_copy(
            edges_hbm.at[pl.ds(e_base + i * _E_CHUNK, _E_CHUNK), :],
            ebufs[slot], sems_in.at[slot])

    def out_copy(i, slot):
        return pltpu.make_async_copy(
            ebufs[slot], out_edges_hbm.at[pl.ds(e_base + i * _E_CHUNK, _E_CHUNK), :],
            sems_out.at[slot])

    outs = [None, None]
    in_copy(0, 0).start()
    for i in range(n_chunks):
        slot = i % 2
        in_copy(i, slot).wait()
        outs[slot] = out_copy(i, slot)
        outs[slot].start()
        if i + 1 < n_chunks:
            nslot = (i + 1) % 2
            if outs[nslot] is not None:
                outs[nslot].wait()
                outs[nslot] = None
            in_copy(i + 1, nslot).start()
    for o in outs:
        if o is not None:
            o.wait()

    total_n_chunks = nodes_hbm.shape[0] // _N_CHUNK
    for k in range((total_n_chunks + _NW - 1) // _NW):
        c = wid + k * _NW

        @pl.when(c < total_n_chunks)
        def _():
            n_base = pl.multiple_of(c * _N_CHUNK, 8)
            pltpu.sync_copy(nodes_hbm.at[pl.ds(n_base, _N_CHUNK), :], nbuf)
            pltpu.sync_copy(nbuf, out_nodes_hbm.at[pl.ds(n_base, _N_CHUNK), :])


def kernel(node_latents_from, node_latents_to, edge_latents, edge_index,
           receivers_count):
    del node_latents_from, edge_index, receivers_count
    mesh = plsc.VectorSubcoreMesh(
        core_axis_name="c", subcore_axis_name="s",
        num_cores=_NC, num_subcores=_NS)
    run = functools.partial(
        pl.kernel,
        out_type=(
            jax.ShapeDtypeStruct(node_latents_to.shape, node_latents_to.dtype),
            jax.ShapeDtypeStruct(edge_latents.shape, edge_latents.dtype),
        ),
        mesh=mesh,
        scratch_types=[
            pltpu.VMEM((_E_CHUNK, 16), jnp.float32),
            pltpu.VMEM((_E_CHUNK, 16), jnp.float32),
            pltpu.VMEM((_N_CHUNK, 128), jnp.float32),
            pltpu.SemaphoreType.DMA((2,)),
            pltpu.SemaphoreType.DMA((2,)),
        ],
    )(_sc_copy_body)
    return run(node_latents_to, edge_latents)


# speedup vs baseline: 17.2758x; 1.0256x over previous
import functools

import jax
import jax.numpy as jnp
from jax import lax
from jax.experimental import pallas as pl
from jax.experimental.pallas import tpu as pltpu
from jax.experimental.pallas import tpu_sc as plsc

_NC = 2
_NS = 16
_NW = _NC * _NS

_RING = 4
_E_CHUNK = 200
_N_CHUNK = 200


def _sc_copy_body(nodes_hbm, edges_hbm, out_nodes_hbm, out_edges_hbm,
                  eb0, eb1, eb2, eb3, nbuf, sems_in, sems_out):
    wid = lax.axis_index("s") * _NC + lax.axis_index("c")
    e_rows = edges_hbm.shape[0] // _NW
    e_base = pl.multiple_of(wid * e_rows, 8)
    n_chunks = e_rows // _E_CHUNK
    ebufs = (eb0, eb1, eb2, eb3)

    def in_copy(i):
        return pltpu.make_async_copy(
            edges_hbm.at[pl.ds(e_base + i * _E_CHUNK, _E_CHUNK), :],
            ebufs[i % _RING], sems_in.at[i % _RING])

    def out_copy(i):
        return pltpu.make_async_copy(
            ebufs[i % _RING],
            out_edges_hbm.at[pl.ds(e_base + i * _E_CHUNK, _E_CHUNK), :],
            sems_out.at[i % _RING])

    for i in range(min(_RING, n_chunks)):
        in_copy(i).start()
    for i in range(n_chunks):
        in_copy(i).wait()
        out_copy(i).start()
        if i + _RING < n_chunks:
            out_copy(i).wait()
            in_copy(i + _RING).start()
    for i in range(max(n_chunks - _RING, 0), n_chunks):
        out_copy(i).wait()

    total_n_chunks = nodes_hbm.shape[0] // _N_CHUNK
    for k in range((total_n_chunks + _NW - 1) // _NW):
        c = wid + k * _NW

        @pl.when(c < total_n_chunks)
        def _():
            n_base = pl.multiple_of(c * _N_CHUNK, 8)
            pltpu.sync_copy(nodes_hbm.at[pl.ds(n_base, _N_CHUNK), :], nbuf)
            pltpu.sync_copy(nbuf, out_nodes_hbm.at[pl.ds(n_base, _N_CHUNK), :])


def kernel(node_latents_from, node_latents_to, edge_latents, edge_index,
           receivers_count):
    del node_latents_from, edge_index, receivers_count
    mesh = plsc.VectorSubcoreMesh(
        core_axis_name="c", subcore_axis_name="s",
        num_cores=_NC, num_subcores=_NS)
    run = functools.partial(
        pl.kernel,
        out_type=(
            jax.ShapeDtypeStruct(node_latents_to.shape, node_latents_to.dtype),
            jax.ShapeDtypeStruct(edge_latents.shape, edge_latents.dtype),
        ),
        mesh=mesh,
        scratch_types=[
            pltpu.VMEM((_E_CHUNK, 16), jnp.float32),
            pltpu.VMEM((_E_CHUNK, 16), jnp.float32),
            pltpu.VMEM((_E_CHUNK, 16), jnp.float32),
            pltpu.VMEM((_E_CHUNK, 16), jnp.float32),
            pltpu.VMEM((_N_CHUNK, 128), jnp.float32),
            pltpu.SemaphoreType.DMA((_RING,)),
            pltpu.SemaphoreType.DMA((_RING,)),
        ],
    )(_sc_copy_body)
    return run(node_latents_to, edge_latents)


# speedup vs baseline: 18.6155x; 1.0775x over previous
import jax
import jax.numpy as jnp
from jax.experimental import pallas as pl
import jax.experimental.pallas.tpu as pltpu

_Q = 8
_S = 10


def _copy_body(nodes_hbm, edges_hbm, out_nodes_hbm, out_edges_hbm,
               nbuf, eb0, eb1, eb2, eb3, eb4, eb5, eb6, eb7,
               sem_n, sems_in, sems_out):
    ebufs = (eb0, eb1, eb2, eb3, eb4, eb5, eb6, eb7)
    e_total = edges_hbm.shape[0]
    rows_per_q = e_total // _Q
    chunk = rows_per_q // _S

    def in_copy(q, s):
        base = q * rows_per_q + s * chunk
        return pltpu.make_async_copy(
            edges_hbm.at[pl.ds(base, chunk), :], ebufs[q], sems_in.at[q])

    def out_copy(q, s):
        base = q * rows_per_q + s * chunk
        return pltpu.make_async_copy(
            ebufs[q], out_edges_hbm.at[pl.ds(base, chunk), :], sems_out.at[q])

    n_in = pltpu.make_async_copy(nodes_hbm, nbuf, sem_n)
    n_in.start()
    for q in range(_Q):
        in_copy(q, 0).start()
    n_in.wait()
    n_out = pltpu.make_async_copy(nbuf, out_nodes_hbm, sem_n)
    n_out.start()
    for s in range(_S):
        for q in range(_Q):
            in_copy(q, s).wait()
            out_copy(q, s).start()
            if s + 1 < _S:
                out_copy(q, s).wait()
                in_copy(q, s + 1).start()
    for q in range(_Q):
        out_copy(q, _S - 1).wait()
    n_out.wait()


def kernel(node_latents_from, node_latents_to, edge_latents, edge_index,
           receivers_count):
    del node_latents_from, edge_index, receivers_count
    n_nodes, d_feat = node_latents_to.shape
    n_edges, d_edge = edge_latents.shape
    chunk = n_edges // _Q // _S
    new_nodes, new_edges = pl.pallas_call(
        _copy_body,
        out_shape=(
            jax.ShapeDtypeStruct(node_latents_to.shape, node_latents_to.dtype),
            jax.ShapeDtypeStruct(edge_latents.shape, edge_latents.dtype),
        ),
        in_specs=[
            pl.BlockSpec(memory_space=pl.ANY),
            pl.BlockSpec(memory_space=pl.ANY),
        ],
        out_specs=(
            pl.BlockSpec(memory_space=pl.ANY),
            pl.BlockSpec(memory_space=pl.ANY),
        ),
        scratch_shapes=(
            [pltpu.VMEM((n_nodes, d_feat), jnp.float32)]
            + [pltpu.VMEM((chunk, d_edge), jnp.float32) for _ in range(_Q)]
            + [pltpu.SemaphoreType.DMA,
               pltpu.SemaphoreType.DMA((_Q,)),
               pltpu.SemaphoreType.DMA((_Q,))]
        ),
    )(node_latents_to, edge_latents)
    return (new_nodes, new_edges)
